# baseline (speedup 1.0000x reference)
import functools

import jax
import jax.numpy as jnp
from jax.experimental import pallas as pl

T = 2048
D = 768
E = 8
K = 2
F = 384
FS = 384

BT = 256


def _moe_block_kernel(x_ref, xb_ref, wg_ref, w1g_ref, w1u_ref, w2_ref,
                      wsg_ref, wsu_ref, wsd_ref, out_ref):
    xf = x_ref[...]
    xb = xb_ref[...]

    logits = jnp.dot(xf, wg_ref[...], preferred_element_type=jnp.float32)
    scores = jax.nn.sigmoid(logits)
    eids = jax.lax.broadcasted_iota(jnp.int32, (BT, E), 1)
    idx1 = jnp.argmax(scores, axis=1)
    v1 = jnp.max(scores, axis=1)
    oh1 = eids == idx1[:, None]
    masked = jnp.where(oh1, -jnp.inf, scores)
    idx2 = jnp.argmax(masked, axis=1)
    v2 = jnp.max(masked, axis=1)
    oh2 = eids == idx2[:, None]
    denom = v1 + v2 + 1e-20
    combine = (oh1 * v1[:, None] + oh2 * v2[:, None]) / denom[:, None]

    sg = jnp.dot(xb, wsg_ref[...], preferred_element_type=jnp.float32)
    su = jnp.dot(xb, wsu_ref[...], preferred_element_type=jnp.float32)
    inter_s = (jax.nn.silu(sg) * su).astype(jnp.bfloat16)
    acc = jnp.dot(inter_s, wsd_ref[...], preferred_element_type=jnp.float32)

    for e in range(E):
        g = jnp.dot(xb, w1g_ref[e], preferred_element_type=jnp.float32)
        u = jnp.dot(xb, w1u_ref[e], preferred_element_type=jnp.float32)
        inter = (jax.nn.silu(g) * u).astype(jnp.bfloat16)
        acc = acc + jnp.dot(inter, w2_ref[e],
                            preferred_element_type=jnp.float32) * combine[:, e:e + 1]

    out_ref[...] = acc


@jax.jit
def kernel(hidden_states, Wg, W1g, W1u, W2, Wsg, Wsu, Wsd):
    bf = jnp.bfloat16
    x_bf = hidden_states.astype(bf)
    grid = (T // BT,)
    return pl.pallas_call(
        _moe_block_kernel,
        grid=grid,
        in_specs=[
            pl.BlockSpec((BT, D), lambda i: (i, 0)),
            pl.BlockSpec((BT, D), lambda i: (i, 0)),
            pl.BlockSpec((D, E), lambda i: (0, 0)),
            pl.BlockSpec((E, D, F), lambda i: (0, 0, 0)),
            pl.BlockSpec((E, D, F), lambda i: (0, 0, 0)),
            pl.BlockSpec((E, F, D), lambda i: (0, 0, 0)),
            pl.BlockSpec((D, FS), lambda i: (0, 0)),
            pl.BlockSpec((D, FS), lambda i: (0, 0)),
            pl.BlockSpec((FS, D), lambda i: (0, 0)),
        ],
        out_specs=pl.BlockSpec((BT, D), lambda i: (i, 0)),
        out_shape=jax.ShapeDtypeStruct((T, D), jnp.float32),
    )(hidden_states, x_bf, Wg, W1g.astype(bf), W1u.astype(bf), W2.astype(bf),
      Wsg.astype(bf), Wsu.astype(bf), Wsd.astype(bf))


# speedup vs baseline: 1.1961x; 1.1961x over previous
import functools

import jax
import jax.numpy as jnp
from jax.experimental import pallas as pl

T = 2048
D = 768
E = 8
K = 2
F = 384
FS = 384

BT = 512


def _moe_block_kernel(x_ref, wg_ref, w1g_ref, w1u_ref, w2_ref,
                      wsg_ref, wsu_ref, wsd_ref, out_ref):
    xf = x_ref[...]
    xb = xf.astype(jnp.bfloat16)

    logits = jnp.dot(xf, wg_ref[...], preferred_element_type=jnp.float32)
    scores = jax.nn.sigmoid(logits)
    eids = jax.lax.broadcasted_iota(jnp.int32, (BT, E), 1)
    idx1 = jnp.argmax(scores, axis=1)
    v1 = jnp.max(scores, axis=1)
    oh1 = eids == idx1[:, None]
    masked = jnp.where(oh1, -jnp.inf, scores)
    idx2 = jnp.argmax(masked, axis=1)
    v2 = jnp.max(masked, axis=1)
    oh2 = eids == idx2[:, None]
    denom = v1 + v2 + 1e-20
    combine = (oh1 * v1[:, None] + oh2 * v2[:, None]) / denom[:, None]

    sg = jnp.dot(xb, wsg_ref[...], preferred_element_type=jnp.float32)
    su = jnp.dot(xb, wsu_ref[...], preferred_element_type=jnp.float32)
    inter_s = (jax.nn.silu(sg) * su).astype(jnp.bfloat16)
    acc = jnp.dot(inter_s, wsd_ref[...], preferred_element_type=jnp.float32)

    for e in range(E):
        g = jnp.dot(xb, w1g_ref[e], preferred_element_type=jnp.float32)
        u = jnp.dot(xb, w1u_ref[e], preferred_element_type=jnp.float32)
        inter = (jax.nn.silu(g) * u).astype(jnp.bfloat16)
        acc = acc + jnp.dot(inter, w2_ref[e],
                            preferred_element_type=jnp.float32) * combine[:, e:e + 1]

    out_ref[...] = acc


@jax.jit
def kernel(hidden_states, Wg, W1g, W1u, W2, Wsg, Wsu, Wsd):
    bf = jnp.bfloat16
    grid = (T // BT,)
    return pl.pallas_call(
        _moe_block_kernel,
        grid=grid,
        in_specs=[
            pl.BlockSpec((BT, D), lambda i: (i, 0)),
            pl.BlockSpec((D, E), lambda i: (0, 0)),
            pl.BlockSpec((E, D, F), lambda i: (0, 0, 0)),
            pl.BlockSpec((E, D, F), lambda i: (0, 0, 0)),
            pl.BlockSpec((E, F, D), lambda i: (0, 0, 0)),
            pl.BlockSpec((D, FS), lambda i: (0, 0)),
            pl.BlockSpec((D, FS), lambda i: (0, 0)),
            pl.BlockSpec((FS, D), lambda i: (0, 0)),
        ],
        out_specs=pl.BlockSpec((BT, D), lambda i: (i, 0)),
        out_shape=jax.ShapeDtypeStruct((T, D), jnp.float32),
    )(hidden_states, Wg, W1g.astype(bf), W1u.astype(bf), W2.astype(bf),
      Wsg.astype(bf), Wsu.astype(bf), Wsd.astype(bf))


# speedup vs baseline: 1.4878x; 1.2439x over previous
import functools

import jax
import jax.numpy as jnp
from jax.experimental import pallas as pl

T = 2048
D = 768
E = 8
K = 2
F = 384
FS = 384

BT = 512


def _moe_block_kernel(x_ref, wg_ref, w1g_ref, w1u_ref, w2_ref,
                      wsg_ref, wsu_ref, wsd_ref, out_ref):
    xf = x_ref[...]
    xb = xf.astype(jnp.bfloat16)

    logits = jnp.dot(xf, wg_ref[...], preferred_element_type=jnp.float32)
    scores = jax.nn.sigmoid(logits)
    eids = jax.lax.broadcasted_iota(jnp.int32, (BT, E), 1)
    idx1 = jnp.argmax(scores, axis=1)
    v1 = jnp.max(scores, axis=1)
    oh1 = eids == idx1[:, None]
    masked = jnp.where(oh1, -jnp.inf, scores)
    idx2 = jnp.argmax(masked, axis=1)
    v2 = jnp.max(masked, axis=1)
    oh2 = eids == idx2[:, None]
    denom = v1 + v2 + 1e-20
    combine = (oh1 * v1[:, None] + oh2 * v2[:, None]) / denom[:, None]

    sg = jnp.dot(xb, wsg_ref[...], preferred_element_type=jnp.float32)
    su = jnp.dot(xb, wsu_ref[...], preferred_element_type=jnp.float32)
    inter_s = (jax.nn.silu(sg) * su).astype(jnp.bfloat16)
    acc = jnp.dot(inter_s, wsd_ref[...], preferred_element_type=jnp.float32)

    for e in range(E):
        g = jnp.dot(xb, w1g_ref[e], preferred_element_type=jnp.float32)
        u = jnp.dot(xb, w1u_ref[e], preferred_element_type=jnp.float32)
        inter = (jax.nn.silu(g) * u).astype(jnp.bfloat16)
        acc = acc + jnp.dot(inter, w2_ref[e],
                            preferred_element_type=jnp.float32) * combine[:, e:e + 1]

    out_ref[...] = acc


@jax.jit
def kernel(hidden_states, Wg, W1g, W1u, W2, Wsg, Wsu, Wsd):
    bf = jnp.bfloat16
    grid = (T // BT,)
    return pl.pallas_call(
        _moe_block_kernel,
        grid=grid,
        in_specs=[
            pl.BlockSpec((BT, D), lambda i: (i, 0)),
            pl.BlockSpec((D, E), lambda i: (0, 0)),
            pl.BlockSpec((E, D, F), lambda i: (0, 0, 0)),
            pl.BlockSpec((E, D, F), lambda i: (0, 0, 0)),
            pl.BlockSpec((E, F, D), lambda i: (0, 0, 0)),
            pl.BlockSpec((D, FS), lambda i: (0, 0)),
            pl.BlockSpec((D, FS), lambda i: (0, 0)),
            pl.BlockSpec((FS, D), lambda i: (0, 0)),
        ],
        out_specs=pl.BlockSpec((BT, D), lambda i: (i, 0)),
        out_shape=jax.ShapeDtypeStruct((T, D), jnp.float32),
    )(hidden_states, Wg, W1g, W1u, W2, Wsg, Wsu, Wsd)


# speedup vs baseline: 1.4895x; 1.0011x over previous
import functools

import jax
import jax.numpy as jnp
from jax.experimental import pallas as pl

T = 2048
D = 768
E = 8
K = 2
F = 384
FS = 384

BT = 1024


def _moe_block_kernel(x_ref, wg_ref, w1g_ref, w1u_ref, w2_ref,
                      wsg_ref, wsu_ref, wsd_ref, out_ref):
    xf = x_ref[...]
    xb = xf.astype(jnp.bfloat16)

    logits = jnp.dot(xf, wg_ref[...], preferred_element_type=jnp.float32)
    scores = jax.nn.sigmoid(logits)
    eids = jax.lax.broadcasted_iota(jnp.int32, (BT, E), 1)
    idx1 = jnp.argmax(scores, axis=1)
    v1 = jnp.max(scores, axis=1)
    oh1 = eids == idx1[:, None]
    masked = jnp.where(oh1, -jnp.inf, scores)
    idx2 = jnp.argmax(masked, axis=1)
    v2 = jnp.max(masked, axis=1)
    oh2 = eids == idx2[:, None]
    denom = v1 + v2 + 1e-20
    combine = (oh1 * v1[:, None] + oh2 * v2[:, None]) / denom[:, None]

    sg = jnp.dot(xb, wsg_ref[...], preferred_element_type=jnp.float32)
    su = jnp.dot(xb, wsu_ref[...], preferred_element_type=jnp.float32)
    inter_s = (jax.nn.silu(sg) * su).astype(jnp.bfloat16)
    acc = jnp.dot(inter_s, wsd_ref[...], preferred_element_type=jnp.float32)

    for e in range(E):
        g = jnp.dot(xb, w1g_ref[e], preferred_element_type=jnp.float32)
        u = jnp.dot(xb, w1u_ref[e], preferred_element_type=jnp.float32)
        inter = (jax.nn.silu(g) * u).astype(jnp.bfloat16)
        acc = acc + jnp.dot(inter, w2_ref[e],
                            preferred_element_type=jnp.float32) * combine[:, e:e + 1]

    out_ref[...] = acc


@jax.jit
def kernel(hidden_states, Wg, W1g, W1u, W2, Wsg, Wsu, Wsd):
    bf = jnp.bfloat16
    grid = (T // BT,)
    return pl.pallas_call(
        _moe_block_kernel,
        grid=grid,
        in_specs=[
            pl.BlockSpec((BT, D), lambda i: (i, 0)),
            pl.BlockSpec((D, E), lambda i: (0, 0)),
            pl.BlockSpec((E, D, F), lambda i: (0, 0, 0)),
            pl.BlockSpec((E, D, F), lambda i: (0, 0, 0)),
            pl.BlockSpec((E, F, D), lambda i: (0, 0, 0)),
            pl.BlockSpec((D, FS), lambda i: (0, 0)),
            pl.BlockSpec((D, FS), lambda i: (0, 0)),
            pl.BlockSpec((FS, D), lambda i: (0, 0)),
        ],
        out_specs=pl.BlockSpec((BT, D), lambda i: (i, 0)),
        out_shape=jax.ShapeDtypeStruct((T, D), jnp.float32),
    )(hidden_states, Wg, W1g, W1u, W2, Wsg, Wsu, Wsd)
